# baseline (speedup 1.0000x reference)
import functools

import jax
import jax.numpy as jnp
from jax import lax
from jax.experimental import pallas as pl
from jax.experimental.pallas import tpu as pltpu
from jax.experimental.pallas import tpu_sc as plsc

N_NODES = 10000
N_EDGES = 160000
D_IN = 256
D_OUT = 256
H = 128
NC = 2
NS = 16
CE = 128
CHUNKS = 80
NBUF = 2
NI = 4
PER_TILE = CHUNKS * CE
E_PAD = PER_TILE * NS
NP = N_NODES + 8
NB = 10240
ROWS_PER_TILE = NB // NS


def _sc_scatter_body(tbl, gidx, didx, zrs, out0, out1, acc, slots, bufs):
    c = lax.axis_index("c")
    s = lax.axis_index("s")
    w = c * NS + s
    gbase = w * PER_TILE
    dbase = s * PER_TILE
    sl = pl.ds(s * ROWS_PER_TILE, ROWS_PER_TILE)
    pltpu.sync_copy(zrs, acc.at[sl])
    plsc.subcore_barrier()

    def fetch_idx(q, j):
        gstg, dstg, isem = slots[q]
        pltpu.async_copy(gidx.at[pl.ds(gbase + j * CE, CE)], gstg, isem)
        pltpu.async_copy(didx.at[pl.ds(dbase + j * CE, CE)], dstg, isem)

    def wait_idx(q):
        gstg, dstg, isem = slots[q]
        pltpu.make_async_copy(gidx.at[pl.ds(gbase, CE)], gstg, isem).wait()
        pltpu.make_async_copy(didx.at[pl.ds(dbase, CE)], dstg, isem).wait()

    for q in range(NI):
        fetch_idx(q, q)
    for b in range(NBUF):
        rows, gsem = bufs[b]
        wait_idx(b)
        pltpu.async_copy(tbl.at[slots[b][0]], rows, gsem)

    def chunk_group(t, carry):
        for k in range(NI):
            j = NI * t + k
            b = k % NBUF
            rows, gsem = bufs[b]
            pltpu.make_async_copy(tbl.at[slots[k][0]], rows, gsem).wait()
            pltpu.sync_copy(rows, acc.at[slots[k][1]], add=True)

            @pl.when(j + NI < CHUNKS)
            def _(k=k, j=j):
                fetch_idx(k, j + NI)

            @pl.when(j + NBUF < CHUNKS)
            def _(k=k, rows=rows, gsem=gsem):
                qn = (k + NBUF) % NI
                wait_idx(qn)
                pltpu.async_copy(tbl.at[slots[qn][0]], rows, gsem)

        return carry

    lax.fori_loop(0, CHUNKS // NI, chunk_group, 0)
    plsc.subcore_barrier()

    @pl.when(c == 0)
    def _():
        pltpu.sync_copy(acc.at[sl], out0.at[sl])

    @pl.when(c == 1)
    def _():
        pltpu.sync_copy(acc.at[sl], out1.at[sl])


_SCRATCH = [pltpu.VMEM_SHARED((NB, H), jnp.float32)]
for _ in range(NI):
    _SCRATCH += [pltpu.VMEM((CE,), jnp.int32),
                 pltpu.VMEM((CE,), jnp.int32),
                 pltpu.SemaphoreType.DMA]
for _ in range(NBUF):
    _SCRATCH += [pltpu.VMEM((CE, H), jnp.float32),
                 pltpu.SemaphoreType.DMA]


@functools.partial(
    pl.kernel,
    out_type=(
        jax.ShapeDtypeStruct((NB, H), jnp.float32),
        jax.ShapeDtypeStruct((NB, H), jnp.float32),
    ),
    mesh=plsc.VectorSubcoreMesh(core_axis_name="c", subcore_axis_name="s"),
    scratch_types=_SCRATCH,
)
def _sc_scatter(tbl, gidx, didx, zrs, out0, out1, acc, *rest):
    slots = tuple(tuple(rest[3 * q:3 * q + 3]) for q in range(NI))
    tail = rest[3 * NI:]
    bufs = tuple(tuple(tail[2 * b:2 * b + 2]) for b in range(NBUF))
    _sc_scatter_body(tbl, gidx, didx, zrs, out0, out1, acc, slots, bufs)


def _loop_mm_body(x_ref, wp_ref, b_ref, o_ref):
    dn = (((1,), (1,)), ((), ()))
    o_ref[...] = lax.dot_general(x_ref[...], wp_ref[...], dn,
                                 preferred_element_type=jnp.float32) + b_ref[...]


def _tc_loop(x, wp, b):
    blk = 1000
    return pl.pallas_call(
        _loop_mm_body,
        grid=(N_NODES // blk,),
        in_specs=[
            pl.BlockSpec((blk, D_IN), lambda i: (i, 0)),
            pl.BlockSpec((D_OUT, D_IN), lambda i: (0, 0)),
            pl.BlockSpec((1, D_OUT), lambda i: (0, 0)),
        ],
        out_specs=pl.BlockSpec((blk, D_OUT), lambda i: (i, 0)),
        out_shape=jax.ShapeDtypeStruct((N_NODES, D_OUT), jnp.float32),
    )(x, wp, b)


def _mm_body(h0_ref, h1_ref, p_ref, wl0_ref, wl1_ref, o_ref):
    dn = (((1,), (1,)), ((), ()))
    acc = lax.dot_general(h0_ref[...], wl0_ref[...], dn,
                          preferred_element_type=jnp.float32)
    acc += lax.dot_general(h1_ref[...], wl1_ref[...], dn,
                           preferred_element_type=jnp.float32)
    o_ref[...] = acc + p_ref[...]


def _tc_linear(h0, h1, part, wl0, wl1):
    blk = 1000
    return pl.pallas_call(
        _mm_body,
        grid=(N_NODES // blk,),
        in_specs=[
            pl.BlockSpec((blk, H), lambda i: (i, 0)),
            pl.BlockSpec((blk, H), lambda i: (i, 0)),
            pl.BlockSpec((blk, D_OUT), lambda i: (i, 0)),
            pl.BlockSpec((D_OUT, H), lambda i: (0, 0)),
            pl.BlockSpec((D_OUT, H), lambda i: (0, 0)),
        ],
        out_specs=pl.BlockSpec((blk, D_OUT), lambda i: (i, 0)),
        out_shape=jax.ShapeDtypeStruct((N_NODES, D_OUT), jnp.float32),
    )(h0, h1, part, wl0, wl1)


def kernel(input_feat, edge_index, W_lin, b_lin, W_loop, b_loop, bias):
    src = edge_index[0].astype(jnp.int32)
    dst = edge_index[1].astype(jnp.int32)
    pad = E_PAD - N_EDGES
    src_p = jnp.concatenate([src, jnp.full((pad,), N_NODES, jnp.int32)])
    dst_p = jnp.concatenate([dst, jnp.full((pad,), N_NODES, jnp.int32)])

    xh = input_feat.reshape(N_NODES, NC, H).transpose(1, 0, 2)
    tbl = jnp.pad(xh, ((0, 0), (0, NP - N_NODES), (0, 0))).reshape(NC * NP, H)

    sp = src_p.reshape(1, NS, PER_TILE)
    gidx = jnp.concatenate([sp, sp + NP], axis=0).reshape(-1)
    didx = dst_p
    zrs = jnp.zeros((ROWS_PER_TILE, H), jnp.float32)

    b = (b_lin + b_loop + bias).reshape(1, D_OUT)
    part = _tc_loop(input_feat, W_loop, b)

    h0, h1 = _sc_scatter(tbl, gidx, didx, zrs)
    h0 = h0[:N_NODES]
    h1 = h1[:N_NODES]

    wl0 = W_lin[:, :H]
    wl1 = W_lin[:, H:]
    return _tc_linear(h0, h1, part, wl0, wl1)


# speedup vs baseline: 1.5578x; 1.5578x over previous
import functools

import jax
import jax.numpy as jnp
from jax import lax
from jax.experimental import pallas as pl
from jax.experimental.pallas import tpu as pltpu
from jax.experimental.pallas import tpu_sc as plsc

N_NODES = 10000
N_EDGES = 160000
D_IN = 256
D_OUT = 256
H = 128
NC = 2
NS = 16
CE = 96
CHUNKS = 105
NBUF = 2
PER_TILE = CHUNKS * CE
E_PAD = PER_TILE * NS
NP = N_NODES + 8
NB = 10240
ROWS_PER_TILE = NB // NS


def _sc_scatter_body(tbl, gidx, didx, zrs, out0, out1, acc, gi, di, bufs):
    c = lax.axis_index("c")
    s = lax.axis_index("s")
    w = c * NS + s
    sl = pl.ds(s * ROWS_PER_TILE, ROWS_PER_TILE)
    pltpu.sync_copy(gidx.at[pl.ds(w * PER_TILE, PER_TILE)], gi)
    pltpu.sync_copy(didx.at[s], di)
    pltpu.sync_copy(zrs, acc.at[sl])
    plsc.subcore_barrier()

    for b in range(NBUF):
        rows, gsem = bufs[b]
        pltpu.async_copy(tbl.at[gi.at[pl.ds(b * CE, CE)]], rows, gsem)

    def chunk_group(t, carry):
        for b in range(NBUF):
            rows, gsem = bufs[b]
            j = NBUF * t + b
            pltpu.make_async_copy(tbl.at[gi.at[pl.ds(0, CE)]], rows,
                                  gsem).wait()
            pltpu.sync_copy(rows, acc.at[di.at[j]], add=True)

            @pl.when(j + NBUF < CHUNKS)
            def _(rows=rows, gsem=gsem, j=j):
                pltpu.async_copy(
                    tbl.at[gi.at[pl.ds((j + NBUF) * CE, CE)]], rows, gsem)

        return carry

    lax.fori_loop(0, CHUNKS // NBUF, chunk_group, 0)
    rows, gsem = bufs[(CHUNKS - 1) % NBUF]
    pltpu.make_async_copy(tbl.at[gi.at[pl.ds(0, CE)]], rows, gsem).wait()
    pltpu.sync_copy(rows, acc.at[di.at[CHUNKS - 1]], add=True)
    plsc.subcore_barrier()

    @pl.when(c == 0)
    def _():
        pltpu.sync_copy(acc.at[sl], out0.at[sl])

    @pl.when(c == 1)
    def _():
        pltpu.sync_copy(acc.at[sl], out1.at[sl])


_SCRATCH = [pltpu.VMEM_SHARED((NB, H), jnp.float32),
            pltpu.VMEM((PER_TILE,), jnp.int32),
            pltpu.VMEM((CHUNKS, CE), jnp.int32)]
for _ in range(NBUF):
    _SCRATCH += [pltpu.VMEM((CE, H), jnp.float32),
                 pltpu.SemaphoreType.DMA]


@functools.partial(
    pl.kernel,
    out_type=(
        jax.ShapeDtypeStruct((NB, H), jnp.float32),
        jax.ShapeDtypeStruct((NB, H), jnp.float32),
    ),
    mesh=plsc.VectorSubcoreMesh(core_axis_name="c", subcore_axis_name="s"),
    scratch_types=_SCRATCH,
)
def _sc_scatter(tbl, gidx, didx, zrs, out0, out1, acc, gi, di, *bufflat):
    bufs = tuple(tuple(bufflat[2 * b:2 * b + 2]) for b in range(NBUF))
    _sc_scatter_body(tbl, gidx, didx, zrs, out0, out1, acc, gi, di, bufs)


def _mm_body(h0_ref, h1_ref, x_ref, wl0_ref, wl1_ref, wp_ref, b_ref, o_ref):
    dn = (((1,), (1,)), ((), ()))
    acc = lax.dot_general(h0_ref[...], wl0_ref[...], dn,
                          preferred_element_type=jnp.float32)
    acc += lax.dot_general(h1_ref[...], wl1_ref[...], dn,
                           preferred_element_type=jnp.float32)
    acc += lax.dot_general(x_ref[...], wp_ref[...], dn,
                           preferred_element_type=jnp.float32)
    o_ref[...] = acc + b_ref[...]


def _tc_linear(h0, h1, x, wl0, wl1, wp, b):
    blk = 1000
    return pl.pallas_call(
        _mm_body,
        grid=(N_NODES // blk,),
        in_specs=[
            pl.BlockSpec((blk, H), lambda i: (i, 0)),
            pl.BlockSpec((blk, H), lambda i: (i, 0)),
            pl.BlockSpec((blk, D_IN), lambda i: (i, 0)),
            pl.BlockSpec((D_OUT, H), lambda i: (0, 0)),
            pl.BlockSpec((D_OUT, H), lambda i: (0, 0)),
            pl.BlockSpec((D_OUT, D_IN), lambda i: (0, 0)),
            pl.BlockSpec((1, D_OUT), lambda i: (0, 0)),
        ],
        out_specs=pl.BlockSpec((blk, D_OUT), lambda i: (i, 0)),
        out_shape=jax.ShapeDtypeStruct((N_NODES, D_OUT), jnp.float32),
    )(h0, h1, x, wl0, wl1, wp, b)


def kernel(input_feat, edge_index, W_lin, b_lin, W_loop, b_loop, bias):
    src = edge_index[0].astype(jnp.int32)
    dst = edge_index[1].astype(jnp.int32)
    pad = E_PAD - N_EDGES
    src_p = jnp.concatenate([src, jnp.full((pad,), N_NODES, jnp.int32)])
    dst_p = jnp.concatenate([dst, jnp.full((pad,), N_NODES, jnp.int32)])

    xh = input_feat.reshape(N_NODES, NC, H).transpose(1, 0, 2)
    tbl = jnp.pad(xh, ((0, 0), (0, NP - N_NODES), (0, 0))).reshape(NC * NP, H)

    sp = src_p.reshape(1, NS, PER_TILE)
    gidx = jnp.concatenate([sp, sp + NP], axis=0).reshape(-1)
    didx = dst_p.reshape(NS, CHUNKS, CE)
    zrs = jnp.zeros((ROWS_PER_TILE, H), jnp.float32)

    h0, h1 = _sc_scatter(tbl, gidx, didx, zrs)
    h0 = h0[:N_NODES]
    h1 = h1[:N_NODES]

    wl0 = W_lin[:, :H]
    wl1 = W_lin[:, H:]
    b = (b_lin + b_loop + bias).reshape(1, D_OUT)
    return _tc_linear(h0, h1, input_feat, wl0, wl1, W_loop, b)


# speedup vs baseline: 1.5811x; 1.0149x over previous
import functools

import jax
import jax.numpy as jnp
from jax import lax
from jax.experimental import pallas as pl
from jax.experimental.pallas import tpu as pltpu
from jax.experimental.pallas import tpu_sc as plsc

N_NODES = 10000
N_EDGES = 160000
D_IN = 256
D_OUT = 256
H = 128
NC = 2
NS = 16
CE = 96
CHUNKS = 105
NBUF = 2
PER_TILE = CHUNKS * CE
E_PAD = PER_TILE * NS
NP = N_NODES + 8
NB = 10240
ROWS_PER_TILE = NB // NS


def _sc_scatter_body(tbl, gidx, didx, zrs, out0, out1, acc, gi, di, bufs):
    c = lax.axis_index("c")
    s = lax.axis_index("s")
    w = c * NS + s
    sl = pl.ds(s * ROWS_PER_TILE, ROWS_PER_TILE)
    pltpu.sync_copy(gidx.at[pl.ds(w * PER_TILE, PER_TILE)], gi)
    pltpu.sync_copy(didx.at[s], di)
    pltpu.sync_copy(zrs, acc.at[sl])
    plsc.subcore_barrier()

    for b in range(NBUF):
        rows, gsem = bufs[b]
        pltpu.async_copy(tbl.at[gi.at[pl.ds(b * CE, CE)]], rows, gsem)

    def chunk_group(t, carry):
        for b in range(NBUF):
            rows, gsem = bufs[b]
            j = NBUF * t + b
            pltpu.make_async_copy(tbl.at[gi.at[pl.ds(0, CE)]], rows,
                                  gsem).wait()
            pltpu.sync_copy(rows, acc.at[di.at[j]], add=True)

            @pl.when(j + NBUF < CHUNKS)
            def _(rows=rows, gsem=gsem, j=j):
                pltpu.async_copy(
                    tbl.at[gi.at[pl.ds((j + NBUF) * CE, CE)]], rows, gsem)

        return carry

    lax.fori_loop(0, CHUNKS // NBUF, chunk_group, 0)
    rows, gsem = bufs[(CHUNKS - 1) % NBUF]
    pltpu.make_async_copy(tbl.at[gi.at[pl.ds(0, CE)]], rows, gsem).wait()
    pltpu.sync_copy(rows, acc.at[di.at[CHUNKS - 1]], add=True)
    plsc.subcore_barrier()

    @pl.when(c == 0)
    def _():
        pltpu.sync_copy(acc.at[sl], out0.at[sl])

    @pl.when(c == 1)
    def _():
        pltpu.sync_copy(acc.at[sl], out1.at[sl])


_SCRATCH = [pltpu.VMEM_SHARED((NB, H), jnp.float32),
            pltpu.VMEM((PER_TILE,), jnp.int32),
            pltpu.VMEM((CHUNKS, CE), jnp.int32)]
for _ in range(NBUF):
    _SCRATCH += [pltpu.VMEM((CE, H), jnp.float32),
                 pltpu.SemaphoreType.DMA]


@functools.partial(
    pl.kernel,
    out_type=(
        jax.ShapeDtypeStruct((NB, H), jnp.float32),
        jax.ShapeDtypeStruct((NB, H), jnp.float32),
    ),
    mesh=plsc.VectorSubcoreMesh(core_axis_name="c", subcore_axis_name="s"),
    scratch_types=_SCRATCH,
)
def _sc_scatter(tbl, gidx, didx, zrs, out0, out1, acc, gi, di, *bufflat):
    bufs = tuple(tuple(bufflat[2 * b:2 * b + 2]) for b in range(NBUF))
    _sc_scatter_body(tbl, gidx, didx, zrs, out0, out1, acc, gi, di, bufs)


def _mm_body(h0_ref, h1_ref, x_ref, wl0_ref, wl1_ref, wp_ref, b_ref, o_ref):
    dn = (((1,), (1,)), ((), ()))
    acc = lax.dot_general(h0_ref[...], wl0_ref[...], dn,
                          preferred_element_type=jnp.float32)
    acc += lax.dot_general(h1_ref[...], wl1_ref[...], dn,
                           preferred_element_type=jnp.float32)
    acc += lax.dot_general(x_ref[...], wp_ref[...], dn,
                           preferred_element_type=jnp.float32)
    o_ref[...] = acc + b_ref[...]


def _tc_linear(h0, h1, x, wl0, wl1, wp, b):
    blk = 1000
    return pl.pallas_call(
        _mm_body,
        grid=(N_NODES // blk,),
        in_specs=[
            pl.BlockSpec((blk, H), lambda i: (i, 0)),
            pl.BlockSpec((blk, H), lambda i: (i, 0)),
            pl.BlockSpec((blk, D_IN), lambda i: (i, 0)),
            pl.BlockSpec((D_OUT, H), lambda i: (0, 0)),
            pl.BlockSpec((D_OUT, H), lambda i: (0, 0)),
            pl.BlockSpec((D_OUT, D_IN), lambda i: (0, 0)),
            pl.BlockSpec((1, D_OUT), lambda i: (0, 0)),
        ],
        out_specs=pl.BlockSpec((blk, D_OUT), lambda i: (i, 0)),
        out_shape=jax.ShapeDtypeStruct((N_NODES, D_OUT), jnp.float32),
    )(h0, h1, x, wl0, wl1, wp, b)


def kernel(input_feat, edge_index, W_lin, b_lin, W_loop, b_loop, bias):
    src = edge_index[0].astype(jnp.int32)
    dst = edge_index[1].astype(jnp.int32)
    pad = E_PAD - N_EDGES
    src_p = jnp.concatenate([src, jnp.zeros((pad,), jnp.int32)])
    dst_p = jnp.concatenate([dst, jnp.full((pad,), N_NODES, jnp.int32)])

    tbl = input_feat.reshape(NC * N_NODES, H)

    sp = 2 * src_p.reshape(1, NS, PER_TILE)
    gidx = jnp.concatenate([sp, sp + 1], axis=0).reshape(-1)
    didx = dst_p.reshape(NS, CHUNKS, CE)
    zrs = jnp.zeros((ROWS_PER_TILE, H), jnp.float32)

    h0, h1 = _sc_scatter(tbl, gidx, didx, zrs)
    h0 = h0[:N_NODES]
    h1 = h1[:N_NODES]

    wl0 = W_lin[:, :H]
    wl1 = W_lin[:, H:]
    b = (b_lin + b_loop + bias).reshape(1, D_OUT)
    return _tc_linear(h0, h1, input_feat, wl0, wl1, W_loop, b)


# speedup vs baseline: 1.6315x; 1.0319x over previous
import functools

import jax
import jax.numpy as jnp
from jax import lax
from jax.experimental import pallas as pl
from jax.experimental.pallas import tpu as pltpu
from jax.experimental.pallas import tpu_sc as plsc

N_NODES = 10000
N_EDGES = 160000
D_IN = 256
D_OUT = 256
H = 128
NC = 2
NS = 16
CE = 96
CHUNKS = 105
NBUF = 2
PER_TILE = CHUNKS * CE
E_PAD = PER_TILE * NS
NP = N_NODES + 8
NB = 10240
ROWS_PER_TILE = NB // NS


def _sc_scatter_body(tbl, gidx, didx, zrs, out0, out1, acc, gi, di, bufs):
    c = lax.axis_index("c")
    s = lax.axis_index("s")
    w = c * NS + s
    sl = pl.ds(s * ROWS_PER_TILE, ROWS_PER_TILE)
    pltpu.sync_copy(gidx.at[pl.ds(w * PER_TILE, PER_TILE)], gi)
    pltpu.sync_copy(didx.at[s], di)
    pltpu.sync_copy(zrs, acc.at[sl])
    plsc.subcore_barrier()

    for b in range(NBUF):
        rows, gsem = bufs[b]
        pltpu.async_copy(tbl.at[gi.at[pl.ds(b * CE, CE)]], rows, gsem)

    def chunk_group(t, carry):
        for b in range(NBUF):
            rows, gsem = bufs[b]
            j = NBUF * t + b
            pltpu.make_async_copy(tbl.at[gi.at[pl.ds(0, CE)]], rows,
                                  gsem).wait()
            pltpu.sync_copy(rows, acc.at[di.at[j]], add=True)

            @pl.when(j + NBUF < CHUNKS)
            def _(rows=rows, gsem=gsem, j=j):
                pltpu.async_copy(
                    tbl.at[gi.at[pl.ds((j + NBUF) * CE, CE)]], rows, gsem)

        return carry

    lax.fori_loop(0, CHUNKS // NBUF, chunk_group, 0)
    rows, gsem = bufs[(CHUNKS - 1) % NBUF]
    pltpu.make_async_copy(tbl.at[gi.at[pl.ds(0, CE)]], rows, gsem).wait()
    pltpu.sync_copy(rows, acc.at[di.at[CHUNKS - 1]], add=True)
    plsc.subcore_barrier()

    @pl.when(c == 0)
    def _():
        pltpu.sync_copy(acc.at[sl], out0.at[sl])

    @pl.when(c == 1)
    def _():
        pltpu.sync_copy(acc.at[sl], out1.at[sl])


_SCRATCH = [pltpu.VMEM_SHARED((NB, H), jnp.float32),
            pltpu.VMEM((PER_TILE,), jnp.int32),
            pltpu.VMEM((CHUNKS, CE), jnp.int32)]
for _ in range(NBUF):
    _SCRATCH += [pltpu.VMEM((CE, H), jnp.float32),
                 pltpu.SemaphoreType.DMA]


@functools.partial(
    pl.kernel,
    out_type=(
        jax.ShapeDtypeStruct((NB, H), jnp.float32),
        jax.ShapeDtypeStruct((NB, H), jnp.float32),
    ),
    mesh=plsc.VectorSubcoreMesh(core_axis_name="c", subcore_axis_name="s"),
    scratch_types=_SCRATCH,
)
def _sc_scatter(tbl, gidx, didx, zrs, out0, out1, acc, gi, di, *bufflat):
    bufs = tuple(tuple(bufflat[2 * b:2 * b + 2]) for b in range(NBUF))
    _sc_scatter_body(tbl, gidx, didx, zrs, out0, out1, acc, gi, di, bufs)


def _mm_body(h0_ref, h1_ref, x_ref, wl0_ref, wl1_ref, wp_ref, b_ref, o_ref):
    dn = (((1,), (1,)), ((), ()))
    acc = lax.dot_general(h0_ref[...], wl0_ref[...], dn,
                          preferred_element_type=jnp.float32)
    acc += lax.dot_general(h1_ref[...], wl1_ref[...], dn,
                           preferred_element_type=jnp.float32)
    acc += lax.dot_general(x_ref[...], wp_ref[...], dn,
                           preferred_element_type=jnp.float32)
    o_ref[...] = acc + b_ref[...]


def _tc_linear(h0, h1, x, wl0, wl1, wp, b):
    blk = 1000
    return pl.pallas_call(
        _mm_body,
        grid=(N_NODES // blk,),
        in_specs=[
            pl.BlockSpec((blk, H), lambda i: (i, 0)),
            pl.BlockSpec((blk, H), lambda i: (i, 0)),
            pl.BlockSpec((blk, D_IN), lambda i: (i, 0)),
            pl.BlockSpec((D_OUT, H), lambda i: (0, 0)),
            pl.BlockSpec((D_OUT, H), lambda i: (0, 0)),
            pl.BlockSpec((D_OUT, D_IN), lambda i: (0, 0)),
            pl.BlockSpec((1, D_OUT), lambda i: (0, 0)),
        ],
        out_specs=pl.BlockSpec((blk, D_OUT), lambda i: (i, 0)),
        out_shape=jax.ShapeDtypeStruct((N_NODES, D_OUT), jnp.float32),
    )(h0, h1, x, wl0, wl1, wp, b)


def kernel(input_feat, edge_index, W_lin, b_lin, W_loop, b_loop, bias):
    src = edge_index[0].astype(jnp.int32)
    dst = edge_index[1].astype(jnp.int32)
    pad = E_PAD - N_EDGES
    src_p = jnp.concatenate([src, jnp.zeros((pad,), jnp.int32)])
    dst_p = jnp.concatenate([dst, jnp.full((pad,), N_NODES, jnp.int32)])

    tbl = input_feat.reshape(NC * N_NODES, H)

    sp = 2 * src_p.reshape(1, NS, PER_TILE)
    gidx = jnp.concatenate([sp, sp + 1], axis=0).reshape(-1)
    didx = dst_p.reshape(NS, CHUNKS, CE)
    zrs = jnp.zeros((ROWS_PER_TILE, H), jnp.float32)

    h0, h1 = _sc_scatter(tbl, gidx, didx, zrs)

    wl0 = W_lin[:, :H]
    wl1 = W_lin[:, H:]
    b = (b_lin + b_loop + bias).reshape(1, D_OUT)
    return _tc_linear(h0, h1, input_feat, wl0, wl1, W_loop, b)
